# baseline (speedup 1.0000x reference)
import jax
import jax.numpy as jnp
from jax.experimental import pallas as pl
from jax.experimental.pallas import tpu as pltpu

N_EMBD = 4096
N_HEAD = 32
BM = 512
NBUF = 4


def _router_kernel(x_hbm, wt_ref, b_ref, gate_ref, idx_ref, xbuf, sems):
    num_tiles = x_hbm.shape[0] // BM

    def start_copy(t):
        slot = jax.lax.rem(t, NBUF)
        pltpu.make_async_copy(
            x_hbm.at[pl.ds(t * BM, BM), :],
            xbuf.at[slot],
            sems.at[slot],
        ).start()

    def wait_copy(t):
        slot = jax.lax.rem(t, NBUF)
        pltpu.make_async_copy(
            x_hbm.at[pl.ds(t * BM, BM), :],
            xbuf.at[slot],
            sems.at[slot],
        ).wait()

    for w in range(NBUF - 1):
        start_copy(w)

    def loop(t, carry):
        @pl.when(t + NBUF - 1 < num_tiles)
        def _():
            start_copy(t + NBUF - 1)

        wait_copy(t)
        slot = jax.lax.rem(t, NBUF)
        xt = xbuf[slot]
        logits = jnp.dot(xt, wt_ref[...], preferred_element_type=jnp.float32)
        logits = logits + b_ref[...]
        lane = jax.lax.broadcasted_iota(jnp.int32, logits.shape, 1)
        neg = jnp.float32(-jnp.inf)
        m1 = jnp.max(logits, axis=1, keepdims=True)
        i1 = jnp.argmax(logits, axis=1).astype(jnp.int32)[:, None]
        l2 = jnp.where(lane == i1, neg, logits)
        m2 = jnp.max(l2, axis=1, keepdims=True)
        i2 = jnp.argmax(l2, axis=1).astype(jnp.int32)[:, None]
        p1 = 1.0 / (1.0 + jnp.exp(m2 - m1))
        p2 = 1.0 - p1
        zero = jnp.zeros_like(logits)
        gate_ref[pl.ds(t * BM, BM), :] = jnp.where(
            lane == i1, p1, jnp.where(lane == i2, p2, zero)
        )
        idx_ref[pl.ds(t * BM, BM), :] = jnp.concatenate([i1, i2], axis=1)
        return carry

    jax.lax.fori_loop(0, num_tiles, loop, 0)


def kernel(x, W, b):
    B, S, D = x.shape
    M = B * S
    xf = x.reshape(M, D)
    wt = W.T
    bp = b.reshape(1, N_HEAD)

    gate, idx = pl.pallas_call(
        _router_kernel,
        in_specs=[
            pl.BlockSpec(memory_space=pl.ANY),
            pl.BlockSpec((D, N_HEAD), lambda: (0, 0)),
            pl.BlockSpec((1, N_HEAD), lambda: (0, 0)),
        ],
        out_specs=[
            pl.BlockSpec((M, N_HEAD), lambda: (0, 0)),
            pl.BlockSpec((M, 2), lambda: (0, 0)),
        ],
        out_shape=[
            jax.ShapeDtypeStruct((M, N_HEAD), jnp.float32),
            jax.ShapeDtypeStruct((M, 2), jnp.int32),
        ],
        scratch_shapes=[
            pltpu.VMEM((NBUF, BM, N_EMBD), jnp.float32),
            pltpu.SemaphoreType.DMA((NBUF,)),
        ],
    )(xf, wt, bp)

    return (gate.reshape(B, S, N_HEAD), idx.reshape(B, S, 2))


# speedup vs baseline: 1.0211x; 1.0211x over previous
import jax
import jax.numpy as jnp
from jax.experimental import pallas as pl
from jax.experimental.pallas import tpu as pltpu

N_EMBD = 4096
N_HEAD = 32
BM = 512
NBUF = 3
NSPLIT = 4
KS = N_EMBD // NSPLIT


def _router_kernel(x_hbm, wt_ref, b_ref, gate_ref, idx_ref, *scratch):
    xbufs = scratch[:NSPLIT]
    sems = scratch[NSPLIT]
    num_tiles = x_hbm.shape[0] // BM

    def copies(t):
        slot = jax.lax.rem(t, NBUF)
        return [
            pltpu.make_async_copy(
                x_hbm.at[pl.ds(t * BM, BM), pl.ds(s * KS, KS)],
                xbufs[s].at[slot],
                sems.at[slot, s],
            )
            for s in range(NSPLIT)
        ]

    def start_copy(t):
        for c in copies(t):
            c.start()

    def wait_copy(t):
        for c in copies(t):
            c.wait()

    for w in range(NBUF - 1):
        start_copy(w)

    def loop(t, carry):
        @pl.when(t + NBUF - 1 < num_tiles)
        def _():
            start_copy(t + NBUF - 1)

        wait_copy(t)
        slot = jax.lax.rem(t, NBUF)
        logits = b_ref[...]
        for s in range(NSPLIT):
            logits = logits + jnp.dot(
                xbufs[s][slot],
                wt_ref[pl.ds(s * KS, KS), :],
                preferred_element_type=jnp.float32,
            )
        lane = jax.lax.broadcasted_iota(jnp.int32, logits.shape, 1)
        neg = jnp.float32(-jnp.inf)
        m1 = jnp.max(logits, axis=1, keepdims=True)
        i1 = jnp.argmax(logits, axis=1).astype(jnp.int32)[:, None]
        l2 = jnp.where(lane == i1, neg, logits)
        m2 = jnp.max(l2, axis=1, keepdims=True)
        i2 = jnp.argmax(l2, axis=1).astype(jnp.int32)[:, None]
        p1 = 1.0 / (1.0 + jnp.exp(m2 - m1))
        p2 = 1.0 - p1
        zero = jnp.zeros_like(logits)
        gate_ref[pl.ds(t * BM, BM), :] = jnp.where(
            lane == i1, p1, jnp.where(lane == i2, p2, zero)
        )
        idx_ref[pl.ds(t * BM, BM), :] = jnp.concatenate([i1, i2], axis=1)
        return carry

    jax.lax.fori_loop(0, num_tiles, loop, 0)


def kernel(x, W, b):
    B, S, D = x.shape
    M = B * S
    xf = x.reshape(M, D)
    wt = W.T
    bp = b.reshape(1, N_HEAD)

    gate, idx = pl.pallas_call(
        _router_kernel,
        in_specs=[
            pl.BlockSpec(memory_space=pl.ANY),
            pl.BlockSpec((D, N_HEAD), lambda: (0, 0)),
            pl.BlockSpec((1, N_HEAD), lambda: (0, 0)),
        ],
        out_specs=[
            pl.BlockSpec((M, N_HEAD), lambda: (0, 0)),
            pl.BlockSpec((M, 2), lambda: (0, 0)),
        ],
        out_shape=[
            jax.ShapeDtypeStruct((M, N_HEAD), jnp.float32),
            jax.ShapeDtypeStruct((M, 2), jnp.int32),
        ],
        scratch_shapes=[
            pltpu.VMEM((NBUF, BM, KS), jnp.float32) for _ in range(NSPLIT)
        ] + [
            pltpu.SemaphoreType.DMA((NBUF, NSPLIT)),
        ],
    )(xf, wt, bp)

    return (gate.reshape(B, S, N_HEAD), idx.reshape(B, S, 2))


# speedup vs baseline: 1.0254x; 1.0042x over previous
import jax
import jax.numpy as jnp
from jax.experimental import pallas as pl
from jax.experimental.pallas import tpu as pltpu

N_EMBD = 4096
N_HEAD = 32
BM = 512


def _router_kernel(x_ref, wt_ref, b_ref, gate_ref, idx_ref):
    logits = jnp.dot(x_ref[...], wt_ref[...], preferred_element_type=jnp.float32)
    logits = logits + b_ref[...]
    lane = jax.lax.broadcasted_iota(jnp.int32, logits.shape, 1)
    neg = jnp.float32(-jnp.inf)
    m1 = jnp.max(logits, axis=1, keepdims=True)
    i1 = jnp.argmax(logits, axis=1).astype(jnp.int32)[:, None]
    l2 = jnp.where(lane == i1, neg, logits)
    m2 = jnp.max(l2, axis=1, keepdims=True)
    i2 = jnp.argmax(l2, axis=1).astype(jnp.int32)[:, None]
    p1 = 1.0 / (1.0 + jnp.exp(m2 - m1))
    p2 = 1.0 - p1
    zero = jnp.zeros_like(logits)
    gate_ref[...] = jnp.where(lane == i1, p1, jnp.where(lane == i2, p2, zero))
    idx_ref[...] = jnp.concatenate([i1, i2], axis=1)


def kernel(x, W, b):
    B, S, D = x.shape
    M = B * S
    xf = x.reshape(M, D)
    wt = W.T
    bp = b.reshape(1, N_HEAD)

    grid = (M // BM,)
    gate, idx = pl.pallas_call(
        _router_kernel,
        grid=grid,
        in_specs=[
            pl.BlockSpec((BM, D), lambda i: (i, 0)),
            pl.BlockSpec((D, N_HEAD), lambda i: (0, 0)),
            pl.BlockSpec((1, N_HEAD), lambda i: (0, 0)),
        ],
        out_specs=[
            pl.BlockSpec((BM, N_HEAD), lambda i: (i, 0)),
            pl.BlockSpec((BM, 2), lambda i: (i, 0)),
        ],
        out_shape=[
            jax.ShapeDtypeStruct((M, N_HEAD), jnp.float32),
            jax.ShapeDtypeStruct((M, 2), jnp.int32),
        ],
        compiler_params=pltpu.CompilerParams(
            dimension_semantics=(pltpu.PARALLEL,),
        ),
    )(xf, wt, bp)

    return (gate.reshape(B, S, N_HEAD), idx.reshape(B, S, 2))
